# baseline (speedup 1.0000x reference)
import functools

import jax
import jax.numpy as jnp
from jax import lax
from jax.experimental import pallas as pl
from jax.experimental.pallas import tpu as pltpu
from jax.experimental.pallas import tpu_sc as plsc

N_NODES = 10000
N_EDGES = 160000
D_IN = 128
D_EDGE = 16
D_OUT = 32
D_EHID = 64

NC = 2
NS = 16
NW = NC * NS
PER_W = N_EDGES // NW
CHUNK = 1000
N_CHUNKS = PER_W // CHUNK
ROWS_PER_TILE = N_NODES // NS

@functools.lru_cache(maxsize=None)
def _sc_mesh():
    return plsc.VectorSubcoreMesh(
        core_axis_name="c", subcore_axis_name="s", num_cores=NC,
        num_subcores=NS,
    )


@functools.lru_cache(maxsize=None)
def _gather_sc_fn():
    @functools.partial(
        pl.kernel,
        out_type=jax.ShapeDtypeStruct((N_EDGES, D_OUT), jnp.float32),
        mesh=_sc_mesh(),
        scratch_types=[
            pltpu.VMEM((CHUNK,), jnp.int32),
            pltpu.VMEM((CHUNK, D_OUT), jnp.float32),
            pltpu.SemaphoreType.DMA,
        ],
        compiler_params=pltpu.CompilerParams(use_tc_tiling_on_sc=False),
    )
    def _gather_sc(h_hbm, src_hbm, out_hbm, idx_v, rows_v, sem):
        wid = lax.axis_index("s") * NC + lax.axis_index("c")
        base = wid * PER_W

        def body(ci, carry):
            off = base + ci * CHUNK
            pltpu.sync_copy(src_hbm.at[pl.ds(off, CHUNK)], idx_v)
            pltpu.async_copy(h_hbm.at[idx_v], rows_v, sem).wait()
            pltpu.sync_copy(rows_v, out_hbm.at[pl.ds(off, CHUNK)])
            return carry

        lax.fori_loop(0, N_CHUNKS, body, 0)

    return _gather_sc


def _gather(h, src):
    return _gather_sc_fn()(h, src)


@functools.lru_cache(maxsize=None)
def _scatter_sc_fn():
    @functools.partial(
        pl.kernel,
        out_type=jax.ShapeDtypeStruct((2 * N_NODES, D_OUT), jnp.float32),
        mesh=_sc_mesh(),
        scratch_types=[
            pltpu.VMEM((CHUNK,), jnp.int32),
            pltpu.VMEM((CHUNK, D_OUT), jnp.float32),
            pltpu.VMEM((ROWS_PER_TILE, D_OUT), jnp.float32),
            pltpu.VMEM_SHARED((N_NODES, D_OUT), jnp.float32),
        ],
        compiler_params=pltpu.CompilerParams(use_tc_tiling_on_sc=False),
    )
    def _scatter_sc(m_hbm, dst_hbm, zeros_hbm, out_hbm, idx_v, rows_v,
                    slab_v, agg_sh):
        cid = lax.axis_index("c")
        sid = lax.axis_index("s")
        wid = sid * NC + cid
        base = wid * PER_W
        slab = sid * ROWS_PER_TILE

        pltpu.sync_copy(zeros_hbm, slab_v)
        pltpu.sync_copy(slab_v, agg_sh.at[pl.ds(slab, ROWS_PER_TILE)])
        plsc.subcore_barrier()

        def body(ci, carry):
            off = base + ci * CHUNK
            pltpu.sync_copy(dst_hbm.at[pl.ds(off, CHUNK)], idx_v)
            pltpu.sync_copy(m_hbm.at[pl.ds(off, CHUNK)], rows_v)
            pltpu.sync_copy(rows_v, agg_sh.at[idx_v], add=True)
            return carry

        lax.fori_loop(0, N_CHUNKS, body, 0)
        plsc.subcore_barrier()

        pltpu.sync_copy(agg_sh.at[pl.ds(slab, ROWS_PER_TILE)], slab_v)
        pltpu.sync_copy(
            slab_v, out_hbm.at[pl.ds(cid * N_NODES + slab, ROWS_PER_TILE)]
        )

    return _scatter_sc


def _scatter(m, dst, zeros):
    return _scatter_sc_fn()(m, dst, zeros)


_PROJ_BLK = 2000


def _proj_body(x_ref, wt_ref, b_ref, o_ref):
    o_ref[...] = jnp.maximum(
        jnp.dot(x_ref[...], wt_ref[...], preferred_element_type=jnp.float32)
        + b_ref[...],
        0.0,
    )


def _proj(x, proj_wt, proj_b):
    return pl.pallas_call(
        _proj_body,
        grid=(N_NODES // _PROJ_BLK,),
        in_specs=[
            pl.BlockSpec((_PROJ_BLK, D_IN), lambda i: (i, 0)),
            pl.BlockSpec((D_IN, D_OUT), lambda i: (0, 0)),
            pl.BlockSpec((1, D_OUT), lambda i: (0, 0)),
        ],
        out_specs=pl.BlockSpec((_PROJ_BLK, D_OUT), lambda i: (i, 0)),
        out_shape=jax.ShapeDtypeStruct((N_NODES, D_OUT), jnp.float32),
    )(x, proj_wt, proj_b)


_EW_BLK = 4000


def _edgenet_body(ea_ref, w1_ref, b1_ref, w2_ref, b2_ref, o_ref):
    a1 = jnp.maximum(
        jnp.dot(ea_ref[...], w1_ref[...], preferred_element_type=jnp.float32)
        + b1_ref[...],
        0.0,
    ).astype(jnp.bfloat16)
    o_ref[...] = (
        jnp.dot(a1, w2_ref[...], preferred_element_type=jnp.float32)
        + b2_ref[...]
    ).astype(jnp.bfloat16)


def _edgenet(edge_attr, ew1t, eb1, ew2tp, eb2p):
    return pl.pallas_call(
        _edgenet_body,
        grid=(N_EDGES // _EW_BLK,),
        in_specs=[
            pl.BlockSpec((_EW_BLK, D_EDGE), lambda i: (i, 0)),
            pl.BlockSpec((D_EDGE, D_EHID), lambda i: (0, 0)),
            pl.BlockSpec((1, D_EHID), lambda i: (0, 0)),
            pl.BlockSpec((D_EHID, D_OUT * D_OUT), lambda i: (0, 0)),
            pl.BlockSpec((1, D_OUT * D_OUT), lambda i: (0, 0)),
        ],
        out_specs=pl.BlockSpec((_EW_BLK, D_OUT * D_OUT), lambda i: (i, 0)),
        out_shape=jax.ShapeDtypeStruct((N_EDGES, D_OUT * D_OUT),
                                       jnp.bfloat16),
    )(edge_attr, ew1t, eb1, ew2tp, eb2p)


_MSG_BLK = 2000


def _msg_body(ewo_ref, hs_ref, fold_ref, m_ref):
    hs = hs_ref[...].astype(jnp.bfloat16)
    hst = jnp.concatenate([hs] * D_OUT, axis=1)
    m_ref[...] = jnp.dot(hst * ewo_ref[...], fold_ref[...],
                         preferred_element_type=jnp.float32)


def _messages(ewo, hs, fold_mat):
    return pl.pallas_call(
        _msg_body,
        grid=(N_EDGES // _MSG_BLK,),
        in_specs=[
            pl.BlockSpec((_MSG_BLK, D_OUT * D_OUT), lambda i: (i, 0)),
            pl.BlockSpec((_MSG_BLK, D_OUT), lambda i: (i, 0)),
            pl.BlockSpec((D_OUT * D_OUT, D_OUT), lambda i: (0, 0)),
        ],
        out_specs=pl.BlockSpec((_MSG_BLK, D_OUT), lambda i: (i, 0)),
        out_shape=jax.ShapeDtypeStruct((N_EDGES, D_OUT), jnp.float32),
    )(ewo, hs, fold_mat)


_GRU_BLK = 2000


def _gru_body(a0_ref, a1_ref, cb_ref, ht_ref, wih_ref, whh_ref, bih_ref,
              bhh_ref, o_ref):
    hc = jnp.maximum(a0_ref[...] + a1_ref[...] + cb_ref[...], 0.0)
    gi = (
        jnp.dot(hc, wih_ref[...], preferred_element_type=jnp.float32)
        + bih_ref[...]
    )
    ht = ht_ref[...]
    gh = (
        jnp.dot(ht, whh_ref[...], preferred_element_type=jnp.float32)
        + bhh_ref[...]
    )
    ir, iz, inn = gi[:, :D_OUT], gi[:, D_OUT:2 * D_OUT], gi[:, 2 * D_OUT:]
    hr, hz, hn = gh[:, :D_OUT], gh[:, D_OUT:2 * D_OUT], gh[:, 2 * D_OUT:]
    r = jax.nn.sigmoid(ir + hr)
    z = jax.nn.sigmoid(iz + hz)
    n = jnp.tanh(inn + r * hn)
    o_ref[...] = (1.0 - z) * n + z * ht


def _gru(parts, conv_b, ht, gru_wiht, gru_whht, gru_bih, gru_bhh):
    return pl.pallas_call(
        _gru_body,
        grid=(N_NODES // _GRU_BLK,),
        in_specs=[
            pl.BlockSpec((_GRU_BLK, D_OUT), lambda i: (i, 0)),
            pl.BlockSpec((_GRU_BLK, D_OUT), lambda i: (i + N_NODES // _GRU_BLK, 0)),
            pl.BlockSpec((1, D_OUT), lambda i: (0, 0)),
            pl.BlockSpec((_GRU_BLK, D_OUT), lambda i: (i, 0)),
            pl.BlockSpec((D_OUT, 3 * D_OUT), lambda i: (0, 0)),
            pl.BlockSpec((D_OUT, 3 * D_OUT), lambda i: (0, 0)),
            pl.BlockSpec((1, 3 * D_OUT), lambda i: (0, 0)),
            pl.BlockSpec((1, 3 * D_OUT), lambda i: (0, 0)),
        ],
        out_specs=pl.BlockSpec((_GRU_BLK, D_OUT), lambda i: (i, 0)),
        out_shape=jax.ShapeDtypeStruct((N_NODES, D_OUT), jnp.float32),
    )(parts, parts, conv_b, ht, gru_wiht, gru_whht, gru_bih, gru_bhh)


def _readout_body(h_ref, wih0_ref, whh0_ref, bih0_ref, bhh0_ref, wih1_ref,
                  whh1_ref, bih1_ref, bhh1_ref, p1_ref, p1b_ref, p2_ref,
                  p2b_ref, o_ref):
    h = h_ref[...]
    q_star = jnp.zeros((1, 2 * D_OUT), jnp.float32)
    h0 = jnp.zeros((1, D_OUT), jnp.float32)
    c0 = jnp.zeros((1, D_OUT), jnp.float32)
    h1 = jnp.zeros((1, D_OUT), jnp.float32)
    c1 = jnp.zeros((1, D_OUT), jnp.float32)

    def lstm(xv, hv, cv, wih, whh, bih, bhh):
        g = (
            jnp.dot(xv, wih, preferred_element_type=jnp.float32) + bih
            + jnp.dot(hv, whh, preferred_element_type=jnp.float32) + bhh
        )
        i = g[:, :D_OUT]
        f = g[:, D_OUT:2 * D_OUT]
        gg = g[:, 2 * D_OUT:3 * D_OUT]
        o = g[:, 3 * D_OUT:]
        c2 = jax.nn.sigmoid(f) * cv + jax.nn.sigmoid(i) * jnp.tanh(gg)
        h2 = jax.nn.sigmoid(o) * jnp.tanh(c2)
        return h2, c2

    for _ in range(3):
        h0, c0 = lstm(q_star, h0, c0, wih0_ref[...], whh0_ref[...],
                      bih0_ref[...], bhh0_ref[...])
        h1, c1 = lstm(h0, h1, c1, wih1_ref[...], whh1_ref[...],
                      bih1_ref[...], bhh1_ref[...])
        q = h1
        e = jnp.sum(h * q, axis=-1, keepdims=True)
        a = jnp.exp(e - jnp.max(e))
        alpha = a / jnp.sum(a)
        readout = jnp.sum(h * alpha, axis=0, keepdims=True)
        q_star = jnp.concatenate([q, readout], axis=-1)

    z = jnp.maximum(
        jnp.dot(q_star, p1_ref[...], preferred_element_type=jnp.float32)
        + p1b_ref[...],
        0.0,
    )
    o_ref[...] = jax.nn.sigmoid(
        jnp.dot(z, p2_ref[...], preferred_element_type=jnp.float32)
        + p2b_ref[...]
    )


def _readout(h, wih0t, whh0t, bih0, bhh0, wih1t, whh1t, bih1, bhh1, p1t, p1b,
             p2t, p2b):
    return pl.pallas_call(
        _readout_body,
        out_shape=jax.ShapeDtypeStruct((1, 1), jnp.float32),
    )(h, wih0t, whh0t, bih0, bhh0, wih1t, whh1t, bih1, bhh1, p1t, p1b, p2t,
      p2b)


def kernel(x, edge_attr, proj_W, proj_b, ew1, eb1, ew2, eb2, conv_b, gru_wih,
           gru_whh, gru_bih, gru_bhh, lstm_wih0, lstm_whh0, lstm_bih0,
           lstm_bhh0, lstm_wih1, lstm_whh1, lstm_bih1, lstm_bhh1, p1_W, p1_b,
           p2_W, p2_b, edge_index):
    src = edge_index[0]
    dst = edge_index[1]
    zeros = jnp.zeros((ROWS_PER_TILE, D_OUT), jnp.float32)

    h = _proj(x, proj_W.T, proj_b.reshape(1, -1))
    ht = h

    perm = (jnp.arange(D_OUT * D_OUT) % D_OUT) * D_OUT + (
        jnp.arange(D_OUT * D_OUT) // D_OUT
    )
    ew2tp = ew2.T[:, perm].astype(jnp.bfloat16)
    eb2p = eb2[perm].reshape(1, -1)
    cols = jnp.arange(D_OUT * D_OUT)
    fold_mat = (
        (cols[:, None] // D_OUT) == jnp.arange(D_OUT)[None, :]
    ).astype(jnp.bfloat16)
    ewo = _edgenet(edge_attr, ew1.T, eb1.reshape(1, -1), ew2tp, eb2p)
    cbr = conv_b.reshape(1, -1)
    wiht = gru_wih.T
    whht = gru_whh.T
    bihr = gru_bih.reshape(1, -1)
    bhhr = gru_bhh.reshape(1, -1)

    for _ in range(3):
        hs = _gather(h, src)
        m = _messages(ewo, hs, fold_mat)
        parts = _scatter(m, dst, zeros)
        ht = _gru(parts, cbr, ht, wiht, whht, bihr, bhhr)
        h = ht

    return _readout(
        h, lstm_wih0.T, lstm_whh0.T, lstm_bih0.reshape(1, -1),
        lstm_bhh0.reshape(1, -1), lstm_wih1.T, lstm_whh1.T,
        lstm_bih1.reshape(1, -1), lstm_bhh1.reshape(1, -1), p1_W.T,
        p1_b.reshape(1, -1), p2_W.T, p2_b.reshape(1, -1)
    )


# speedup vs baseline: 1.0181x; 1.0181x over previous
import functools

import jax
import jax.numpy as jnp
from jax import lax
from jax.experimental import pallas as pl
from jax.experimental.pallas import tpu as pltpu
from jax.experimental.pallas import tpu_sc as plsc

N_NODES = 10000
N_EDGES = 160000
D_IN = 128
D_EDGE = 16
D_OUT = 32
D_EHID = 64

NC = 2
NS = 16
NW = NC * NS
PER_W = N_EDGES // NW
CHUNK = 1000
N_CHUNKS = PER_W // CHUNK
ROWS_PER_TILE = N_NODES // NS

@functools.lru_cache(maxsize=None)
def _sc_mesh():
    return plsc.VectorSubcoreMesh(
        core_axis_name="c", subcore_axis_name="s", num_cores=NC,
        num_subcores=NS,
    )


@functools.lru_cache(maxsize=None)
def _gather_sc_fn():
    @functools.partial(
        pl.kernel,
        out_type=jax.ShapeDtypeStruct((N_EDGES, D_OUT), jnp.float32),
        mesh=_sc_mesh(),
        scratch_types=[
            pltpu.VMEM((CHUNK,), jnp.int32),
            pltpu.VMEM((CHUNK, D_OUT), jnp.float32),
            pltpu.SemaphoreType.DMA,
        ],
        compiler_params=pltpu.CompilerParams(use_tc_tiling_on_sc=False),
    )
    def _gather_sc(h_hbm, src_hbm, out_hbm, idx_v, rows_v, sem):
        wid = lax.axis_index("s") * NC + lax.axis_index("c")
        base = wid * PER_W

        def body(ci, carry):
            off = base + ci * CHUNK
            pltpu.sync_copy(src_hbm.at[pl.ds(off, CHUNK)], idx_v)
            pltpu.async_copy(h_hbm.at[idx_v], rows_v, sem).wait()
            pltpu.sync_copy(rows_v, out_hbm.at[pl.ds(off, CHUNK)])
            return carry

        lax.fori_loop(0, N_CHUNKS, body, 0)

    return _gather_sc


def _gather(h, src):
    return _gather_sc_fn()(h, src)


@functools.lru_cache(maxsize=None)
def _scatter_sc_fn():
    @functools.partial(
        pl.kernel,
        out_type=jax.ShapeDtypeStruct((2 * N_NODES, D_OUT), jnp.float32),
        mesh=_sc_mesh(),
        scratch_types=[
            pltpu.VMEM((CHUNK,), jnp.int32),
            pltpu.VMEM((CHUNK, D_OUT), jnp.float32),
            pltpu.VMEM((ROWS_PER_TILE, D_OUT), jnp.float32),
            pltpu.VMEM_SHARED((N_NODES, D_OUT), jnp.float32),
        ],
        compiler_params=pltpu.CompilerParams(use_tc_tiling_on_sc=False),
    )
    def _scatter_sc(m_hbm, dst_hbm, zeros_hbm, out_hbm, idx_v, rows_v,
                    slab_v, agg_sh):
        cid = lax.axis_index("c")
        sid = lax.axis_index("s")
        wid = sid * NC + cid
        base = wid * PER_W
        slab = sid * ROWS_PER_TILE

        pltpu.sync_copy(zeros_hbm, slab_v)
        pltpu.sync_copy(slab_v, agg_sh.at[pl.ds(slab, ROWS_PER_TILE)])
        plsc.subcore_barrier()

        def body(ci, carry):
            off = base + ci * CHUNK
            pltpu.sync_copy(dst_hbm.at[pl.ds(off, CHUNK)], idx_v)
            pltpu.sync_copy(m_hbm.at[pl.ds(off, CHUNK)], rows_v)
            pltpu.sync_copy(rows_v, agg_sh.at[idx_v], add=True)
            return carry

        lax.fori_loop(0, N_CHUNKS, body, 0)
        plsc.subcore_barrier()

        pltpu.sync_copy(agg_sh.at[pl.ds(slab, ROWS_PER_TILE)], slab_v)
        pltpu.sync_copy(
            slab_v, out_hbm.at[pl.ds(cid * N_NODES + slab, ROWS_PER_TILE)]
        )

    return _scatter_sc


def _scatter(m, dst, zeros):
    return _scatter_sc_fn()(m, dst, zeros)


_PROJ_BLK = 2000


def _proj_body(x_ref, wt_ref, b_ref, o_ref):
    o_ref[...] = jnp.maximum(
        jnp.dot(x_ref[...], wt_ref[...], preferred_element_type=jnp.float32)
        + b_ref[...],
        0.0,
    )


def _proj(x, proj_wt, proj_b):
    return pl.pallas_call(
        _proj_body,
        grid=(N_NODES // _PROJ_BLK,),
        in_specs=[
            pl.BlockSpec((_PROJ_BLK, D_IN), lambda i: (i, 0)),
            pl.BlockSpec((D_IN, D_OUT), lambda i: (0, 0)),
            pl.BlockSpec((1, D_OUT), lambda i: (0, 0)),
        ],
        out_specs=pl.BlockSpec((_PROJ_BLK, D_OUT), lambda i: (i, 0)),
        out_shape=jax.ShapeDtypeStruct((N_NODES, D_OUT), jnp.float32),
    )(x, proj_wt, proj_b)


_EH_BLK = 8000


def _edgehid_body(ea_ref, w1_ref, b1_ref, o_ref):
    o_ref[...] = jnp.maximum(
        jnp.dot(ea_ref[...], w1_ref[...], preferred_element_type=jnp.float32)
        + b1_ref[...],
        0.0,
    ).astype(jnp.bfloat16)


def _edgehid(edge_attr, ew1t, eb1):
    return pl.pallas_call(
        _edgehid_body,
        grid=(N_EDGES // _EH_BLK,),
        in_specs=[
            pl.BlockSpec((_EH_BLK, D_EDGE), lambda i: (i, 0)),
            pl.BlockSpec((D_EDGE, D_EHID), lambda i: (0, 0)),
            pl.BlockSpec((1, D_EHID), lambda i: (0, 0)),
        ],
        out_specs=pl.BlockSpec((_EH_BLK, D_EHID), lambda i: (i, 0)),
        out_shape=jax.ShapeDtypeStruct((N_EDGES, D_EHID), jnp.bfloat16),
    )(edge_attr, ew1t, eb1)


_MSG_BLK = 2000


def _msg_body(a1_ref, hs_ref, w2_ref, b2_ref, fold_ref, m_ref):
    ewo = (
        jnp.dot(a1_ref[...], w2_ref[...],
                preferred_element_type=jnp.float32)
        + b2_ref[...]
    ).astype(jnp.bfloat16)
    hs = hs_ref[...].astype(jnp.bfloat16)
    hst = jnp.concatenate([hs] * D_OUT, axis=1)
    m_ref[...] = jnp.dot(hst * ewo, fold_ref[...],
                         preferred_element_type=jnp.float32)


def _messages(a1, hs, ew2tp, eb2p, fold_mat):
    return pl.pallas_call(
        _msg_body,
        grid=(N_EDGES // _MSG_BLK,),
        in_specs=[
            pl.BlockSpec((_MSG_BLK, D_EHID), lambda i: (i, 0)),
            pl.BlockSpec((_MSG_BLK, D_OUT), lambda i: (i, 0)),
            pl.BlockSpec((D_EHID, D_OUT * D_OUT), lambda i: (0, 0)),
            pl.BlockSpec((1, D_OUT * D_OUT), lambda i: (0, 0)),
            pl.BlockSpec((D_OUT * D_OUT, D_OUT), lambda i: (0, 0)),
        ],
        out_specs=pl.BlockSpec((_MSG_BLK, D_OUT), lambda i: (i, 0)),
        out_shape=jax.ShapeDtypeStruct((N_EDGES, D_OUT), jnp.float32),
    )(a1, hs, ew2tp, eb2p, fold_mat)


_GRU_BLK = 2000


def _gru_body(a0_ref, a1_ref, cb_ref, ht_ref, wih_ref, whh_ref, bih_ref,
              bhh_ref, o_ref):
    hc = jnp.maximum(a0_ref[...] + a1_ref[...] + cb_ref[...], 0.0)
    gi = (
        jnp.dot(hc, wih_ref[...], preferred_element_type=jnp.float32)
        + bih_ref[...]
    )
    ht = ht_ref[...]
    gh = (
        jnp.dot(ht, whh_ref[...], preferred_element_type=jnp.float32)
        + bhh_ref[...]
    )
    ir, iz, inn = gi[:, :D_OUT], gi[:, D_OUT:2 * D_OUT], gi[:, 2 * D_OUT:]
    hr, hz, hn = gh[:, :D_OUT], gh[:, D_OUT:2 * D_OUT], gh[:, 2 * D_OUT:]
    r = jax.nn.sigmoid(ir + hr)
    z = jax.nn.sigmoid(iz + hz)
    n = jnp.tanh(inn + r * hn)
    o_ref[...] = (1.0 - z) * n + z * ht


def _gru(parts, conv_b, ht, gru_wiht, gru_whht, gru_bih, gru_bhh):
    return pl.pallas_call(
        _gru_body,
        grid=(N_NODES // _GRU_BLK,),
        in_specs=[
            pl.BlockSpec((_GRU_BLK, D_OUT), lambda i: (i, 0)),
            pl.BlockSpec((_GRU_BLK, D_OUT), lambda i: (i + N_NODES // _GRU_BLK, 0)),
            pl.BlockSpec((1, D_OUT), lambda i: (0, 0)),
            pl.BlockSpec((_GRU_BLK, D_OUT), lambda i: (i, 0)),
            pl.BlockSpec((D_OUT, 3 * D_OUT), lambda i: (0, 0)),
            pl.BlockSpec((D_OUT, 3 * D_OUT), lambda i: (0, 0)),
            pl.BlockSpec((1, 3 * D_OUT), lambda i: (0, 0)),
            pl.BlockSpec((1, 3 * D_OUT), lambda i: (0, 0)),
        ],
        out_specs=pl.BlockSpec((_GRU_BLK, D_OUT), lambda i: (i, 0)),
        out_shape=jax.ShapeDtypeStruct((N_NODES, D_OUT), jnp.float32),
    )(parts, parts, conv_b, ht, gru_wiht, gru_whht, gru_bih, gru_bhh)


def _readout_body(h_ref, wih0_ref, whh0_ref, bih0_ref, bhh0_ref, wih1_ref,
                  whh1_ref, bih1_ref, bhh1_ref, p1_ref, p1b_ref, p2_ref,
                  p2b_ref, o_ref):
    h = h_ref[...]
    q_star = jnp.zeros((1, 2 * D_OUT), jnp.float32)
    h0 = jnp.zeros((1, D_OUT), jnp.float32)
    c0 = jnp.zeros((1, D_OUT), jnp.float32)
    h1 = jnp.zeros((1, D_OUT), jnp.float32)
    c1 = jnp.zeros((1, D_OUT), jnp.float32)

    def lstm(xv, hv, cv, wih, whh, bih, bhh):
        g = (
            jnp.dot(xv, wih, preferred_element_type=jnp.float32) + bih
            + jnp.dot(hv, whh, preferred_element_type=jnp.float32) + bhh
        )
        i = g[:, :D_OUT]
        f = g[:, D_OUT:2 * D_OUT]
        gg = g[:, 2 * D_OUT:3 * D_OUT]
        o = g[:, 3 * D_OUT:]
        c2 = jax.nn.sigmoid(f) * cv + jax.nn.sigmoid(i) * jnp.tanh(gg)
        h2 = jax.nn.sigmoid(o) * jnp.tanh(c2)
        return h2, c2

    for _ in range(3):
        h0, c0 = lstm(q_star, h0, c0, wih0_ref[...], whh0_ref[...],
                      bih0_ref[...], bhh0_ref[...])
        h1, c1 = lstm(h0, h1, c1, wih1_ref[...], whh1_ref[...],
                      bih1_ref[...], bhh1_ref[...])
        q = h1
        e = jnp.sum(h * q, axis=-1, keepdims=True)
        a = jnp.exp(e - jnp.max(e))
        alpha = a / jnp.sum(a)
        readout = jnp.sum(h * alpha, axis=0, keepdims=True)
        q_star = jnp.concatenate([q, readout], axis=-1)

    z = jnp.maximum(
        jnp.dot(q_star, p1_ref[...], preferred_element_type=jnp.float32)
        + p1b_ref[...],
        0.0,
    )
    o_ref[...] = jax.nn.sigmoid(
        jnp.dot(z, p2_ref[...], preferred_element_type=jnp.float32)
        + p2b_ref[...]
    )


def _readout(h, wih0t, whh0t, bih0, bhh0, wih1t, whh1t, bih1, bhh1, p1t, p1b,
             p2t, p2b):
    return pl.pallas_call(
        _readout_body,
        out_shape=jax.ShapeDtypeStruct((1, 1), jnp.float32),
    )(h, wih0t, whh0t, bih0, bhh0, wih1t, whh1t, bih1, bhh1, p1t, p1b, p2t,
      p2b)


def kernel(x, edge_attr, proj_W, proj_b, ew1, eb1, ew2, eb2, conv_b, gru_wih,
           gru_whh, gru_bih, gru_bhh, lstm_wih0, lstm_whh0, lstm_bih0,
           lstm_bhh0, lstm_wih1, lstm_whh1, lstm_bih1, lstm_bhh1, p1_W, p1_b,
           p2_W, p2_b, edge_index):
    src = edge_index[0]
    dst = edge_index[1]
    zeros = jnp.zeros((ROWS_PER_TILE, D_OUT), jnp.float32)

    h = _proj(x, proj_W.T, proj_b.reshape(1, -1))
    ht = h

    a1 = _edgehid(edge_attr, ew1.T, eb1.reshape(1, -1))
    perm = (jnp.arange(D_OUT * D_OUT) % D_OUT) * D_OUT + (
        jnp.arange(D_OUT * D_OUT) // D_OUT
    )
    ew2tp = ew2.T[:, perm].astype(jnp.bfloat16)
    eb2p = eb2[perm].reshape(1, -1)
    cols = jnp.arange(D_OUT * D_OUT)
    fold_mat = (
        (cols[:, None] // D_OUT) == jnp.arange(D_OUT)[None, :]
    ).astype(jnp.bfloat16)
    cbr = conv_b.reshape(1, -1)
    wiht = gru_wih.T
    whht = gru_whh.T
    bihr = gru_bih.reshape(1, -1)
    bhhr = gru_bhh.reshape(1, -1)

    for _ in range(3):
        hs = _gather(h, src)
        m = _messages(a1, hs, ew2tp, eb2p, fold_mat)
        parts = _scatter(m, dst, zeros)
        ht = _gru(parts, cbr, ht, wiht, whht, bihr, bhhr)
        h = ht

    return _readout(
        h, lstm_wih0.T, lstm_whh0.T, lstm_bih0.reshape(1, -1),
        lstm_bhh0.reshape(1, -1), lstm_wih1.T, lstm_whh1.T,
        lstm_bih1.reshape(1, -1), lstm_bhh1.reshape(1, -1), p1_W.T,
        p1_b.reshape(1, -1), p2_W.T, p2_b.reshape(1, -1)
    )
